# baseline (speedup 1.0000x reference)
import functools

import jax
import jax.numpy as jnp
from jax import lax
from jax.experimental import pallas as pl
from jax.experimental.pallas import tpu as pltpu
from jax.experimental.pallas import tpu_sc as plsc

N = 10000
E_RAW = 320000
D_IN = 128
D_H = 256
HALF = 128

CHE = 64
E2P = 360448
PAD = E2P - (E_RAW + N)
CR = E2P // CHE
RPT = CR // 16
RPW = CR // 32
GS = 8
NBUF = 4

ROWS = 10240
STR = ROWS // 16
HSTR = STR // 2

DROWS = 10240
DSTR = DROWS // 16

NB = 1000
G = N // NB

@functools.cache
def _get_sc_deg():
    return functools.partial(
        pl.kernel,
        out_type=jax.ShapeDtypeStruct((2, DROWS), jnp.float32),
        mesh=plsc.VectorSubcoreMesh(core_axis_name="c", subcore_axis_name="s"),
        scratch_types=[
            pltpu.VMEM((GS, CHE), jnp.int32),
            pltpu.VMEM((CHE,), jnp.float32),
            pltpu.VMEM((DSTR,), jnp.float32),
            pltpu.VMEM_SHARED((DROWS,), jnp.float32),
        ],
    )(_sc_deg_body)


def _sc_deg_body(dst_h, deg_out, dstb, ones, zbuf, accum):
    c = lax.axis_index("c")
    s = lax.axis_index("s")
    w = s * 2 + c

    def fill(i, _):
        zbuf[pl.ds(i * 16, 16)] = jnp.zeros((16,), jnp.float32)
        ones[pl.ds((i % 4) * 16, 16)] = jnp.ones((16,), jnp.float32)
        return 0

    lax.fori_loop(0, DSTR // 16, fill, 0)
    pltpu.sync_copy(zbuf, accum.at[pl.ds(s * DSTR, DSTR)])
    plsc.subcore_barrier()

    def group(gi, _):
        pltpu.sync_copy(dst_h.at[pl.ds(w * RPW + gi * GS, GS)], dstb)

        def chunk(j, _):
            pltpu.sync_copy(ones, accum.at[dstb.at[j]], add=True)
            return 0

        lax.fori_loop(0, GS, chunk, 0)
        return 0

    lax.fori_loop(0, RPW // GS, group, 0)
    plsc.subcore_barrier()
    pltpu.sync_copy(accum.at[pl.ds(s * DSTR, DSTR)], zbuf)
    pltpu.sync_copy(zbuf, deg_out.at[c, pl.ds(s * DSTR, DSTR)])


@functools.cache
def _get_sc_conv():
    return functools.partial(
        pl.kernel,
        out_type=(jax.ShapeDtypeStruct((ROWS, HALF), jnp.float32),
                  jax.ShapeDtypeStruct((ROWS, HALF), jnp.float32)),
        mesh=plsc.VectorSubcoreMesh(core_axis_name="c", subcore_axis_name="s"),
        scratch_types=[
            pltpu.VMEM((GS, CHE), jnp.int32),
            pltpu.VMEM((GS, CHE), jnp.int32),
            pltpu.VMEM((CHE, HALF), jnp.float32),
            pltpu.VMEM((CHE, HALF), jnp.float32),
            pltpu.VMEM((CHE, HALF), jnp.float32),
            pltpu.VMEM((CHE, HALF), jnp.float32),
            pltpu.VMEM_SHARED((ROWS, HALF), jnp.float32),
            pltpu.SemaphoreType.DMA,
            pltpu.SemaphoreType.DMA,
            pltpu.SemaphoreType.DMA,
            pltpu.SemaphoreType.DMA,
            pltpu.SemaphoreType.DMA,
            pltpu.SemaphoreType.DMA,
            pltpu.SemaphoreType.DMA,
            pltpu.SemaphoreType.DMA,
        ],
    )(_sc_conv_body)


def _sc_conv_body(hs_lo, hs_hi, src_h, dst_h, out_lo, out_hi,
                  srcb, dstb, rows0, rows1, rows2, rows3, accum,
                  gsem0, gsem1, gsem2, gsem3, ssem0, ssem1, ssem2, ssem3):
    c = lax.axis_index("c")
    s = lax.axis_index("s")
    rows = (rows0, rows1, rows2, rows3)
    gsem = (gsem0, gsem1, gsem2, gsem3)
    ssem = (ssem0, ssem1, ssem2, ssem3)

    def fill(i, _):
        rows0[i // 8, pl.ds((i % 8) * 16, 16)] = jnp.zeros((16,), jnp.float32)
        return 0

    lax.fori_loop(0, CHE * (HALF // 16), fill, 0)
    base = s * STR
    for p in range(STR // CHE):
        pltpu.sync_copy(rows0, accum.at[pl.ds(base + p * CHE, CHE)])
    plsc.subcore_barrier()

    def run(table, out):
        def gwait(b):
            pltpu.make_async_copy(table.at[pl.ds(0, CHE)], rows[b],
                                  gsem[b]).wait()

        def swait(b):
            pltpu.make_async_copy(rows[b], out.at[pl.ds(0, CHE)],
                                  ssem[b]).wait()

        def group(gi, _):
            g0 = s * RPT + gi * GS
            pltpu.sync_copy(src_h.at[pl.ds(g0, GS)], srcb)
            pltpu.sync_copy(dst_h.at[pl.ds(g0, GS)], dstb)
            for j in range(NBUF - 1):
                pltpu.async_copy(table.at[srcb.at[j]], rows[j], gsem[j])
            for j in range(GS):
                b = j % NBUF
                if j + NBUF - 1 < GS:
                    b3 = (j + NBUF - 1) % NBUF
                    if j >= 1:
                        swait(b3)
                    pltpu.async_copy(table.at[srcb.at[j + NBUF - 1]],
                                     rows[b3], gsem[b3])
                gwait(b)
                pltpu.async_copy(rows[b], accum.at[dstb.at[j]], ssem[b],
                                 add=True)
            for b in range(NBUF):
                swait(b)
            return 0

        lax.fori_loop(0, RPT // GS, group, 0)
        plsc.subcore_barrier()
        for p in range(STR // CHE):
            pltpu.sync_copy(accum.at[pl.ds(base + p * CHE, CHE)], rows0)
            pltpu.sync_copy(rows0, out.at[pl.ds(base + p * CHE, CHE)])

    @pl.when(c == 0)
    def _():
        run(hs_lo, out_lo)

    @pl.when(c == 1)
    def _():
        run(hs_hi, out_hi)


def _dense_in_body(x_ref, degp_ref, Win_ref, bin_ref, Wg1_ref,
                   h0_ref, hslo_ref, hshi_ref, dinv_ref):
    deg = degp_ref[:, 0:1] + degp_ref[:, 1:2]
    dinv = lax.rsqrt(deg)
    h0 = jnp.maximum(
        jnp.dot(x_ref[...], Win_ref[...], preferred_element_type=jnp.float32)
        + bin_ref[...], 0.0)
    hs = jnp.dot(h0, Wg1_ref[...], preferred_element_type=jnp.float32) * dinv
    h0_ref[...] = h0
    hslo_ref[...] = hs[:, :HALF]
    hshi_ref[...] = hs[:, HALF:]
    dinv_ref[...] = dinv


def _post_ln(agglo_ref, agghi_ref, dinv_ref, hid_ref, bg_ref, g_ref, be_ref):
    dinv = dinv_ref[...]
    agg = jnp.concatenate([agglo_ref[...], agghi_ref[...]], axis=1) * dinv
    agg = agg + bg_ref[...]
    mu = jnp.mean(agg, axis=1, keepdims=True)
    var = jnp.mean((agg - mu) ** 2, axis=1, keepdims=True)
    y = (agg - mu) * lax.rsqrt(var + 1e-5) * g_ref[...] + be_ref[...]
    return jnp.maximum(y, 0.0) + hid_ref[...], dinv


def _dense_mid_body(agglo_ref, agghi_ref, dinv_ref, hid_ref,
                    bg_ref, g_ref, be_ref, Wg_ref,
                    h_ref, hslo_ref, hshi_ref):
    h, dinv = _post_ln(agglo_ref, agghi_ref, dinv_ref, hid_ref,
                       bg_ref, g_ref, be_ref)
    h_ref[...] = h
    hs = jnp.dot(h, Wg_ref[...], preferred_element_type=jnp.float32) * dinv
    hslo_ref[...] = hs[:, :HALF]
    hshi_ref[...] = hs[:, HALF:]


def _dense_out_body(agglo_ref, agghi_ref, dinv_ref, hid_ref,
                    bg_ref, g_ref, be_ref,
                    Wf1_ref, bf1_ref, Wf2_ref, bf2_ref, out_ref):
    h, _ = _post_ln(agglo_ref, agghi_ref, dinv_ref, hid_ref,
                    bg_ref, g_ref, be_ref)
    t = jnp.maximum(
        jnp.dot(h, Wf1_ref[...], preferred_element_type=jnp.float32)
        + bf1_ref[...], 0.0)
    out_ref[...] = (jnp.dot(t, Wf2_ref[...], preferred_element_type=jnp.float32)
                    + bf2_ref[...])


def _row_spec(w):
    return pl.BlockSpec((NB, w), lambda g: (g, 0))


def _full_spec(h, w):
    return pl.BlockSpec((h, w), lambda g: (0, 0))


_dense_in = pl.pallas_call(
    _dense_in_body,
    grid=(G,),
    in_specs=[_row_spec(D_IN), _row_spec(2), _full_spec(D_IN, D_H),
              _full_spec(1, D_H), _full_spec(D_H, D_H)],
    out_specs=[_row_spec(D_H), _row_spec(HALF), _row_spec(HALF), _row_spec(1)],
    out_shape=[jax.ShapeDtypeStruct((N, D_H), jnp.float32),
               jax.ShapeDtypeStruct((N, HALF), jnp.float32),
               jax.ShapeDtypeStruct((N, HALF), jnp.float32),
               jax.ShapeDtypeStruct((N, 1), jnp.float32)],
)

_dense_mid = pl.pallas_call(
    _dense_mid_body,
    grid=(G,),
    in_specs=[_row_spec(HALF), _row_spec(HALF), _row_spec(1), _row_spec(D_H),
              _full_spec(1, D_H), _full_spec(1, D_H), _full_spec(1, D_H),
              _full_spec(D_H, D_H)],
    out_specs=[_row_spec(D_H), _row_spec(HALF), _row_spec(HALF)],
    out_shape=[jax.ShapeDtypeStruct((N, D_H), jnp.float32),
               jax.ShapeDtypeStruct((N, HALF), jnp.float32),
               jax.ShapeDtypeStruct((N, HALF), jnp.float32)],
)

_dense_out = pl.pallas_call(
    _dense_out_body,
    grid=(G,),
    in_specs=[_row_spec(HALF), _row_spec(HALF), _row_spec(1), _row_spec(D_H),
              _full_spec(1, D_H), _full_spec(1, D_H), _full_spec(1, D_H),
              _full_spec(D_H, HALF), _full_spec(1, HALF),
              _full_spec(HALF, 1), _full_spec(1, 1)],
    out_specs=_row_spec(1),
    out_shape=jax.ShapeDtypeStruct((N, 1), jnp.float32),
)


def kernel(x, edge_index, W_in, b_in, Wg1, bg1, g1, be1,
           Wg2, bg2, g2, be2, Wf1, bf1, Wf2, bf2):
    ei = edge_index.astype(jnp.int32)
    loop = jnp.arange(N, dtype=jnp.int32)
    src = jnp.concatenate([ei[0], loop, jnp.zeros((PAD,), jnp.int32)])
    dst = jnp.concatenate([ei[1], loop, jnp.full((PAD,), N, jnp.int32)])
    src = src.reshape(CR, CHE)
    dst = dst.reshape(CR, CHE)

    degp = _get_sc_deg()(dst).T

    h0, hs1lo, hs1hi, dinv = _dense_in(
        x, degp, W_in, b_in.reshape(1, D_H), Wg1)
    a1lo, a1hi = _get_sc_conv()(hs1lo, hs1hi, src, dst)
    h1, hs2lo, hs2hi = _dense_mid(
        a1lo, a1hi, dinv, h0, bg1.reshape(1, D_H), g1.reshape(1, D_H),
        be1.reshape(1, D_H), Wg2)
    a2lo, a2hi = _get_sc_conv()(hs2lo, hs2hi, src, dst)
    out2 = _dense_out(
        a2lo, a2hi, dinv, h1, bg2.reshape(1, D_H), g2.reshape(1, D_H),
        be2.reshape(1, D_H), Wf1, bf1.reshape(1, HALF),
        Wf2, bf2.reshape(1, 1))
    return out2[:, 0]


# speedup vs baseline: 4.0377x; 4.0377x over previous
import functools

import jax
import jax.numpy as jnp
from jax import lax
from jax.experimental import pallas as pl
from jax.experimental.pallas import tpu as pltpu
from jax.experimental.pallas import tpu_sc as plsc

N = 10000
E_RAW = 320000
D_IN = 128
D_H = 256
HALF = 128

CHE = 128
E2P = 360448
PAD = E2P - (E_RAW + N)
CR = E2P // CHE
RPT = CR // 16
RPW = CR // 32
GS = 8
NBUF = 2

ROWS = 10240
STR = ROWS // 16

DROWS = 10240
DSTR = DROWS // 16

NB = 1024
G = ROWS // NB

@functools.cache
def _get_sc_deg():
    return functools.partial(
        pl.kernel,
        out_type=jax.ShapeDtypeStruct((2, DROWS), jnp.float32),
        mesh=plsc.VectorSubcoreMesh(core_axis_name="c", subcore_axis_name="s"),
        scratch_types=[
            pltpu.VMEM((GS, CHE), jnp.int32),
            pltpu.VMEM((CHE,), jnp.float32),
            pltpu.VMEM((DSTR,), jnp.float32),
            pltpu.VMEM_SHARED((DROWS,), jnp.float32),
        ],
    )(_sc_deg_body)


def _sc_deg_body(dst_h, deg_out, dstb, ones, zbuf, accum):
    c = lax.axis_index("c")
    s = lax.axis_index("s")
    w = s * 2 + c

    def fill(i, _):
        zbuf[pl.ds(i * 16, 16)] = jnp.zeros((16,), jnp.float32)
        ones[pl.ds((i % (CHE // 16)) * 16, 16)] = jnp.ones((16,), jnp.float32)
        return 0

    lax.fori_loop(0, DSTR // 16, fill, 0)
    pltpu.sync_copy(zbuf, accum.at[pl.ds(s * DSTR, DSTR)])
    plsc.subcore_barrier()

    def group(gi, _):
        pltpu.sync_copy(dst_h.at[pl.ds(w * RPW + gi * GS, GS)], dstb)

        def chunk(j, _):
            pltpu.sync_copy(ones, accum.at[dstb.at[j]], add=True)
            return 0

        lax.fori_loop(0, GS, chunk, 0)
        return 0

    lax.fori_loop(0, RPW // GS, group, 0)
    plsc.subcore_barrier()
    pltpu.sync_copy(accum.at[pl.ds(s * DSTR, DSTR)], zbuf)
    pltpu.sync_copy(zbuf, deg_out.at[c, pl.ds(s * DSTR, DSTR)])


@functools.cache
def _get_sc_msg():
    return functools.partial(
        pl.kernel,
        out_type=(jax.ShapeDtypeStruct((E2P, HALF), jnp.float32),
                  jax.ShapeDtypeStruct((E2P, HALF), jnp.float32)),
        mesh=plsc.VectorSubcoreMesh(core_axis_name="c", subcore_axis_name="s"),
        scratch_types=[
            pltpu.VMEM((GS, CHE), jnp.int32),
            pltpu.VMEM((CHE, HALF), jnp.float32),
            pltpu.VMEM((CHE, HALF), jnp.float32),
            pltpu.VMEM_SHARED((ROWS, HALF), jnp.float32),
            pltpu.SemaphoreType.DMA,
            pltpu.SemaphoreType.DMA,
            pltpu.SemaphoreType.DMA,
            pltpu.SemaphoreType.DMA,
        ],
    )(_sc_msg_body)


def _sc_msg_body(hs_lo, hs_hi, src_h, msg_lo, msg_hi,
                 srcb, rows0, rows1, table_s,
                 gsem0, gsem1, ssem0, ssem1):
    c = lax.axis_index("c")
    s = lax.axis_index("s")
    rows = (rows0, rows1)
    gsem = (gsem0, gsem1)
    ssem = (ssem0, ssem1)
    base = s * STR

    def run(tbl, msg):
        for p in range(STR // CHE):
            pltpu.sync_copy(tbl.at[pl.ds(base + p * CHE, CHE)], rows0)
            pltpu.sync_copy(rows0, table_s.at[pl.ds(base + p * CHE, CHE)])
        plsc.subcore_barrier()

        def gwait(b):
            pltpu.make_async_copy(table_s.at[pl.ds(0, CHE)], rows[b],
                                  gsem[b]).wait()

        def swait(b):
            pltpu.make_async_copy(rows[b], msg.at[pl.ds(0, CHE)],
                                  ssem[b]).wait()

        def group(gi, _):
            g0 = s * RPT + gi * GS
            pltpu.sync_copy(src_h.at[pl.ds(g0, GS)], srcb)
            for j in range(NBUF - 1):
                pltpu.async_copy(table_s.at[srcb.at[j]], rows[j], gsem[j])
            for j in range(GS):
                b = j % NBUF
                if j + NBUF - 1 < GS:
                    b3 = (j + NBUF - 1) % NBUF
                    if j >= 1:
                        swait(b3)
                    pltpu.async_copy(table_s.at[srcb.at[j + NBUF - 1]],
                                     rows[b3], gsem[b3])
                gwait(b)
                pltpu.async_copy(rows[b], msg.at[pl.ds((g0 + j) * CHE, CHE)],
                                 ssem[b])
            for b in range(NBUF):
                swait(b)
            return 0

        lax.fori_loop(0, RPT // GS, group, 0)

    @pl.when(c == 0)
    def _():
        run(hs_lo, msg_lo)

    @pl.when(c == 1)
    def _():
        run(hs_hi, msg_hi)


@functools.cache
def _get_sc_agg():
    return functools.partial(
        pl.kernel,
        out_type=(jax.ShapeDtypeStruct((ROWS, HALF), jnp.float32),
                  jax.ShapeDtypeStruct((ROWS, HALF), jnp.float32)),
        mesh=plsc.VectorSubcoreMesh(core_axis_name="c", subcore_axis_name="s"),
        scratch_types=[
            pltpu.VMEM((GS, CHE), jnp.int32),
            pltpu.VMEM((CHE, HALF), jnp.float32),
            pltpu.VMEM((CHE, HALF), jnp.float32),
            pltpu.VMEM_SHARED((ROWS, HALF), jnp.float32),
            pltpu.SemaphoreType.DMA,
            pltpu.SemaphoreType.DMA,
            pltpu.SemaphoreType.DMA,
            pltpu.SemaphoreType.DMA,
        ],
    )(_sc_agg_body)


def _sc_agg_body(msg_lo, msg_hi, dst_h, out_lo, out_hi,
                 dstb, rows0, rows1, accum,
                 gsem0, gsem1, ssem0, ssem1):
    c = lax.axis_index("c")
    s = lax.axis_index("s")
    rows = (rows0, rows1)
    gsem = (gsem0, gsem1)
    ssem = (ssem0, ssem1)
    base = s * STR

    def fill(i, _):
        rows0[i // (HALF // 16), pl.ds((i % (HALF // 16)) * 16, 16)] = (
            jnp.zeros((16,), jnp.float32))
        return 0

    lax.fori_loop(0, CHE * (HALF // 16), fill, 0)
    for p in range(STR // CHE):
        pltpu.sync_copy(rows0, accum.at[pl.ds(base + p * CHE, CHE)])
    plsc.subcore_barrier()

    def run(msg, out):
        def gwait(b):
            pltpu.make_async_copy(msg.at[pl.ds(0, CHE)], rows[b],
                                  gsem[b]).wait()

        def swait(b):
            pltpu.make_async_copy(rows[b], out.at[pl.ds(0, CHE)],
                                  ssem[b]).wait()

        def group(gi, _):
            g0 = s * RPT + gi * GS
            pltpu.sync_copy(dst_h.at[pl.ds(g0, GS)], dstb)
            for j in range(NBUF - 1):
                pltpu.async_copy(msg.at[pl.ds((g0 + j) * CHE, CHE)], rows[j],
                                 gsem[j])
            for j in range(GS):
                b = j % NBUF
                if j + NBUF - 1 < GS:
                    b3 = (j + NBUF - 1) % NBUF
                    if j >= 1:
                        swait(b3)
                    pltpu.async_copy(
                        msg.at[pl.ds((g0 + j + NBUF - 1) * CHE, CHE)],
                        rows[b3], gsem[b3])
                gwait(b)
                pltpu.async_copy(rows[b], accum.at[dstb.at[j]], ssem[b],
                                 add=True)
            for b in range(NBUF):
                swait(b)
            return 0

        lax.fori_loop(0, RPT // GS, group, 0)
        plsc.subcore_barrier()
        for p in range(STR // CHE):
            pltpu.sync_copy(accum.at[pl.ds(base + p * CHE, CHE)], rows0)
            pltpu.sync_copy(rows0, out.at[pl.ds(base + p * CHE, CHE)])

    @pl.when(c == 0)
    def _():
        run(msg_lo, out_lo)

    @pl.when(c == 1)
    def _():
        run(msg_hi, out_hi)


def _dense_in_body(x_ref, degp_ref, Win_ref, bin_ref, Wg1_ref,
                   h0_ref, hslo_ref, hshi_ref, dinv_ref):
    deg = degp_ref[:, 0:1] + degp_ref[:, 1:2]
    dinv = lax.rsqrt(deg)
    h0 = jnp.maximum(
        jnp.dot(x_ref[...], Win_ref[...], preferred_element_type=jnp.float32)
        + bin_ref[...], 0.0)
    hs = jnp.dot(h0, Wg1_ref[...], preferred_element_type=jnp.float32) * dinv
    h0_ref[...] = h0
    hslo_ref[...] = hs[:, :HALF]
    hshi_ref[...] = hs[:, HALF:]
    dinv_ref[...] = dinv


def _post_ln(agglo_ref, agghi_ref, dinv_ref, hid_ref, bg_ref, g_ref, be_ref):
    dinv = dinv_ref[...]
    agg = jnp.concatenate([agglo_ref[...], agghi_ref[...]], axis=1) * dinv
    agg = agg + bg_ref[...]
    mu = jnp.mean(agg, axis=1, keepdims=True)
    var = jnp.mean((agg - mu) ** 2, axis=1, keepdims=True)
    y = (agg - mu) * lax.rsqrt(var + 1e-5) * g_ref[...] + be_ref[...]
    return jnp.maximum(y, 0.0) + hid_ref[...], dinv


def _dense_mid_body(agglo_ref, agghi_ref, dinv_ref, hid_ref,
                    bg_ref, g_ref, be_ref, Wg_ref,
                    h_ref, hslo_ref, hshi_ref):
    h, dinv = _post_ln(agglo_ref, agghi_ref, dinv_ref, hid_ref,
                       bg_ref, g_ref, be_ref)
    h_ref[...] = h
    hs = jnp.dot(h, Wg_ref[...], preferred_element_type=jnp.float32) * dinv
    hslo_ref[...] = hs[:, :HALF]
    hshi_ref[...] = hs[:, HALF:]


def _dense_out_body(agglo_ref, agghi_ref, dinv_ref, hid_ref,
                    bg_ref, g_ref, be_ref,
                    Wf1_ref, bf1_ref, Wf2_ref, bf2_ref, out_ref):
    h, _ = _post_ln(agglo_ref, agghi_ref, dinv_ref, hid_ref,
                    bg_ref, g_ref, be_ref)
    t = jnp.maximum(
        jnp.dot(h, Wf1_ref[...], preferred_element_type=jnp.float32)
        + bf1_ref[...], 0.0)
    out_ref[...] = (jnp.dot(t, Wf2_ref[...], preferred_element_type=jnp.float32)
                    + bf2_ref[...])


def _row_spec(w):
    return pl.BlockSpec((NB, w), lambda g: (g, 0))


def _full_spec(h, w):
    return pl.BlockSpec((h, w), lambda g: (0, 0))


_half_shape = jax.ShapeDtypeStruct((ROWS, HALF), jnp.float32)

_dense_in = pl.pallas_call(
    _dense_in_body,
    grid=(G,),
    in_specs=[_row_spec(D_IN), _row_spec(2), _full_spec(D_IN, D_H),
              _full_spec(1, D_H), _full_spec(D_H, D_H)],
    out_specs=[_row_spec(D_H), _row_spec(HALF), _row_spec(HALF), _row_spec(1)],
    out_shape=[jax.ShapeDtypeStruct((N, D_H), jnp.float32),
               _half_shape, _half_shape,
               jax.ShapeDtypeStruct((N, 1), jnp.float32)],
)

_dense_mid = pl.pallas_call(
    _dense_mid_body,
    grid=(G,),
    in_specs=[_row_spec(HALF), _row_spec(HALF), _row_spec(1), _row_spec(D_H),
              _full_spec(1, D_H), _full_spec(1, D_H), _full_spec(1, D_H),
              _full_spec(D_H, D_H)],
    out_specs=[_row_spec(D_H), _row_spec(HALF), _row_spec(HALF)],
    out_shape=[jax.ShapeDtypeStruct((N, D_H), jnp.float32),
               _half_shape, _half_shape],
)

_dense_out = pl.pallas_call(
    _dense_out_body,
    grid=(G,),
    in_specs=[_row_spec(HALF), _row_spec(HALF), _row_spec(1), _row_spec(D_H),
              _full_spec(1, D_H), _full_spec(1, D_H), _full_spec(1, D_H),
              _full_spec(D_H, HALF), _full_spec(1, HALF),
              _full_spec(HALF, 1), _full_spec(1, 1)],
    out_specs=_row_spec(1),
    out_shape=jax.ShapeDtypeStruct((N, 1), jnp.float32),
)


def _conv_agg(hs_lo, hs_hi, src, dst):
    mlo, mhi = _get_sc_msg()(hs_lo, hs_hi, src)
    return _get_sc_agg()(mlo, mhi, dst)


def kernel(x, edge_index, W_in, b_in, Wg1, bg1, g1, be1,
           Wg2, bg2, g2, be2, Wf1, bf1, Wf2, bf2):
    ei = edge_index.astype(jnp.int32)
    loop = jnp.arange(N, dtype=jnp.int32)
    src = jnp.concatenate([ei[0], loop, jnp.zeros((PAD,), jnp.int32)])
    dst = jnp.concatenate([ei[1], loop, jnp.full((PAD,), N, jnp.int32)])
    src = src.reshape(CR, CHE)
    dst = dst.reshape(CR, CHE)

    degp = _get_sc_deg()(dst).T

    h0, hs1lo, hs1hi, dinv = _dense_in(
        x, degp, W_in, b_in.reshape(1, D_H), Wg1)
    a1lo, a1hi = _conv_agg(hs1lo, hs1hi, src, dst)
    h1, hs2lo, hs2hi = _dense_mid(
        a1lo, a1hi, dinv, h0, bg1.reshape(1, D_H), g1.reshape(1, D_H),
        be1.reshape(1, D_H), Wg2)
    a2lo, a2hi = _conv_agg(hs2lo, hs2hi, src, dst)
    out2 = _dense_out(
        a2lo, a2hi, dinv, h1, bg2.reshape(1, D_H), g2.reshape(1, D_H),
        be2.reshape(1, D_H), Wf1, bf1.reshape(1, HALF),
        Wf2, bf2.reshape(1, 1))
    return out2[:, 0]


# speedup vs baseline: 4.4437x; 1.1006x over previous
import functools

import jax
import jax.numpy as jnp
from jax import lax
from jax.experimental import pallas as pl
from jax.experimental.pallas import tpu as pltpu
from jax.experimental.pallas import tpu_sc as plsc

N = 10000
E_RAW = 320000
D_IN = 128
D_H = 256
HALF = 128

CHE = 128
E2P = 327680
PAD = E2P - E_RAW
CR = E2P // CHE
RPT = CR // 16
RPW = CR // 32
GS = 8
NBUF = 2

ROWS = 10240
STR = ROWS // 16

DROWS = 10240
DSTR = DROWS // 16

NB = 1024
G = ROWS // NB

@functools.cache
def _get_sc_deg():
    return functools.partial(
        pl.kernel,
        out_type=jax.ShapeDtypeStruct((2, DROWS), jnp.float32),
        mesh=plsc.VectorSubcoreMesh(core_axis_name="c", subcore_axis_name="s"),
        scratch_types=[
            pltpu.VMEM((GS, CHE), jnp.int32),
            pltpu.VMEM((CHE,), jnp.float32),
            pltpu.VMEM((DSTR,), jnp.float32),
            pltpu.VMEM_SHARED((DROWS,), jnp.float32),
        ],
    )(_sc_deg_body)


def _sc_deg_body(dst_h, deg_out, dstb, ones, zbuf, accum):
    c = lax.axis_index("c")
    s = lax.axis_index("s")
    w = s * 2 + c

    def fill(i, _):
        zbuf[pl.ds(i * 16, 16)] = jnp.zeros((16,), jnp.float32)
        ones[pl.ds((i % (CHE // 16)) * 16, 16)] = jnp.ones((16,), jnp.float32)
        return 0

    lax.fori_loop(0, DSTR // 16, fill, 0)
    pltpu.sync_copy(zbuf, accum.at[pl.ds(s * DSTR, DSTR)])
    plsc.subcore_barrier()

    def group(gi, _):
        pltpu.sync_copy(dst_h.at[pl.ds(w * RPW + gi * GS, GS)], dstb)

        def chunk(j, _):
            pltpu.sync_copy(ones, accum.at[dstb.at[j]], add=True)
            return 0

        lax.fori_loop(0, GS, chunk, 0)
        return 0

    lax.fori_loop(0, RPW // GS, group, 0)
    plsc.subcore_barrier()
    pltpu.sync_copy(accum.at[pl.ds(s * DSTR, DSTR)], zbuf)
    pltpu.sync_copy(zbuf, deg_out.at[c, pl.ds(s * DSTR, DSTR)])


@functools.cache
def _get_sc_msg():
    return functools.partial(
        pl.kernel,
        out_type=(jax.ShapeDtypeStruct((E2P, HALF), jnp.float32),
                  jax.ShapeDtypeStruct((E2P, HALF), jnp.float32)),
        mesh=plsc.VectorSubcoreMesh(core_axis_name="c", subcore_axis_name="s"),
        scratch_types=[
            pltpu.VMEM((GS, CHE), jnp.int32),
            pltpu.VMEM((CHE, HALF), jnp.float32),
            pltpu.VMEM((CHE, HALF), jnp.float32),
            pltpu.VMEM_SHARED((ROWS, HALF), jnp.float32),
            pltpu.SemaphoreType.DMA,
            pltpu.SemaphoreType.DMA,
            pltpu.SemaphoreType.DMA,
            pltpu.SemaphoreType.DMA,
        ],
    )(_sc_msg_body)


def _sc_msg_body(hs_lo, hs_hi, src_h, msg_lo, msg_hi,
                 srcb, rows0, rows1, table_s,
                 gsem0, gsem1, ssem0, ssem1):
    c = lax.axis_index("c")
    s = lax.axis_index("s")
    rows = (rows0, rows1)
    gsem = (gsem0, gsem1)
    ssem = (ssem0, ssem1)
    base = s * STR

    def run(tbl, msg):
        for p in range(STR // CHE):
            pltpu.sync_copy(tbl.at[pl.ds(base + p * CHE, CHE)], rows0)
            pltpu.sync_copy(rows0, table_s.at[pl.ds(base + p * CHE, CHE)])
        plsc.subcore_barrier()

        def gwait(b):
            pltpu.make_async_copy(table_s.at[pl.ds(0, CHE)], rows[b],
                                  gsem[b]).wait()

        def swait(b):
            pltpu.make_async_copy(rows[b], msg.at[pl.ds(0, CHE)],
                                  ssem[b]).wait()

        def group(gi, _):
            g0 = s * RPT + gi * GS
            pltpu.sync_copy(src_h.at[pl.ds(g0, GS)], srcb)
            for j in range(NBUF - 1):
                pltpu.async_copy(table_s.at[srcb.at[j]], rows[j], gsem[j])
            for j in range(GS):
                b = j % NBUF
                if j + NBUF - 1 < GS:
                    b3 = (j + NBUF - 1) % NBUF
                    if j >= 1:
                        swait(b3)
                    pltpu.async_copy(table_s.at[srcb.at[j + NBUF - 1]],
                                     rows[b3], gsem[b3])
                gwait(b)
                pltpu.async_copy(rows[b], msg.at[pl.ds((g0 + j) * CHE, CHE)],
                                 ssem[b])
            for b in range(NBUF):
                swait(b)
            return 0

        lax.fori_loop(0, RPT // GS, group, 0)

    @pl.when(c == 0)
    def _():
        run(hs_lo, msg_lo)

    @pl.when(c == 1)
    def _():
        run(hs_hi, msg_hi)


@functools.cache
def _get_sc_agg():
    return functools.partial(
        pl.kernel,
        out_type=(jax.ShapeDtypeStruct((ROWS, HALF), jnp.float32),
                  jax.ShapeDtypeStruct((ROWS, HALF), jnp.float32)),
        mesh=plsc.VectorSubcoreMesh(core_axis_name="c", subcore_axis_name="s"),
        scratch_types=[
            pltpu.VMEM((GS, CHE), jnp.int32),
            pltpu.VMEM((CHE, HALF), jnp.float32),
            pltpu.VMEM((CHE, HALF), jnp.float32),
            pltpu.VMEM_SHARED((ROWS, HALF), jnp.float32),
            pltpu.SemaphoreType.DMA,
            pltpu.SemaphoreType.DMA,
            pltpu.SemaphoreType.DMA,
            pltpu.SemaphoreType.DMA,
        ],
    )(_sc_agg_body)


def _sc_agg_body(msg_lo, msg_hi, dst_h, out_lo, out_hi,
                 dstb, rows0, rows1, accum,
                 gsem0, gsem1, ssem0, ssem1):
    c = lax.axis_index("c")
    s = lax.axis_index("s")
    rows = (rows0, rows1)
    gsem = (gsem0, gsem1)
    ssem = (ssem0, ssem1)
    base = s * STR

    def fill(i, _):
        rows0[i // (HALF // 16), pl.ds((i % (HALF // 16)) * 16, 16)] = (
            jnp.zeros((16,), jnp.float32))
        return 0

    lax.fori_loop(0, CHE * (HALF // 16), fill, 0)
    for p in range(STR // CHE):
        pltpu.sync_copy(rows0, accum.at[pl.ds(base + p * CHE, CHE)])
    plsc.subcore_barrier()

    def run(msg, out):
        def gwait(b):
            pltpu.make_async_copy(msg.at[pl.ds(0, CHE)], rows[b],
                                  gsem[b]).wait()

        def swait(b):
            pltpu.make_async_copy(rows[b], out.at[pl.ds(0, CHE)],
                                  ssem[b]).wait()

        def group(gi, _):
            g0 = s * RPT + gi * GS
            pltpu.sync_copy(dst_h.at[pl.ds(g0, GS)], dstb)
            for j in range(NBUF - 1):
                pltpu.async_copy(msg.at[pl.ds((g0 + j) * CHE, CHE)], rows[j],
                                 gsem[j])
            for j in range(GS):
                b = j % NBUF
                if j + NBUF - 1 < GS:
                    b3 = (j + NBUF - 1) % NBUF
                    if j >= 1:
                        swait(b3)
                    pltpu.async_copy(
                        msg.at[pl.ds((g0 + j + NBUF - 1) * CHE, CHE)],
                        rows[b3], gsem[b3])
                gwait(b)
                pltpu.async_copy(rows[b], accum.at[dstb.at[j]], ssem[b],
                                 add=True)
            for b in range(NBUF):
                swait(b)
            return 0

        lax.fori_loop(0, RPT // GS, group, 0)
        plsc.subcore_barrier()
        for p in range(STR // CHE):
            pltpu.sync_copy(accum.at[pl.ds(base + p * CHE, CHE)], rows0)
            pltpu.sync_copy(rows0, out.at[pl.ds(base + p * CHE, CHE)])

    @pl.when(c == 0)
    def _():
        run(msg_lo, out_lo)

    @pl.when(c == 1)
    def _():
        run(msg_hi, out_hi)


def _dense_in_body(x_ref, degp_ref, Win_ref, bin_ref, Wg1_ref,
                   h0_ref, hslo_ref, hshi_ref, dinv_ref):
    deg = degp_ref[:, 0:1] + degp_ref[:, 1:2] + 1.0
    dinv = lax.rsqrt(deg)
    h0 = jnp.maximum(
        jnp.dot(x_ref[...], Win_ref[...], preferred_element_type=jnp.float32)
        + bin_ref[...], 0.0)
    hs = jnp.dot(h0, Wg1_ref[...], preferred_element_type=jnp.float32) * dinv
    h0_ref[...] = h0
    hslo_ref[...] = hs[:, :HALF]
    hshi_ref[...] = hs[:, HALF:]
    dinv_ref[...] = dinv


def _post_ln(agglo_ref, agghi_ref, hsl_ref, hsh_ref,
             dinv_ref, hid_ref, bg_ref, g_ref, be_ref):
    dinv = dinv_ref[...]
    agg = jnp.concatenate([agglo_ref[...] + hsl_ref[...],
                           agghi_ref[...] + hsh_ref[...]], axis=1) * dinv
    agg = agg + bg_ref[...]
    mu = jnp.mean(agg, axis=1, keepdims=True)
    var = jnp.mean((agg - mu) ** 2, axis=1, keepdims=True)
    y = (agg - mu) * lax.rsqrt(var + 1e-5) * g_ref[...] + be_ref[...]
    return jnp.maximum(y, 0.0) + hid_ref[...], dinv


def _dense_mid_body(agglo_ref, agghi_ref, hsl_ref, hsh_ref, dinv_ref,
                    hid_ref, bg_ref, g_ref, be_ref, Wg_ref,
                    h_ref, hslo_ref, hshi_ref):
    h, dinv = _post_ln(agglo_ref, agghi_ref, hsl_ref, hsh_ref, dinv_ref,
                       hid_ref, bg_ref, g_ref, be_ref)
    h_ref[...] = h
    hs = jnp.dot(h, Wg_ref[...], preferred_element_type=jnp.float32) * dinv
    hslo_ref[...] = hs[:, :HALF]
    hshi_ref[...] = hs[:, HALF:]


def _dense_out_body(agglo_ref, agghi_ref, hsl_ref, hsh_ref, dinv_ref,
                    hid_ref, bg_ref, g_ref, be_ref,
                    Wf1_ref, bf1_ref, Wf2_ref, bf2_ref, out_ref):
    h, _ = _post_ln(agglo_ref, agghi_ref, hsl_ref, hsh_ref, dinv_ref,
                    hid_ref, bg_ref, g_ref, be_ref)
    t = jnp.maximum(
        jnp.dot(h, Wf1_ref[...], preferred_element_type=jnp.float32)
        + bf1_ref[...], 0.0)
    out_ref[...] = (jnp.dot(t, Wf2_ref[...], preferred_element_type=jnp.float32)
                    + bf2_ref[...])


def _row_spec(w):
    return pl.BlockSpec((NB, w), lambda g: (g, 0))


def _full_spec(h, w):
    return pl.BlockSpec((h, w), lambda g: (0, 0))


_half_shape = jax.ShapeDtypeStruct((ROWS, HALF), jnp.float32)

_dense_in = pl.pallas_call(
    _dense_in_body,
    grid=(G,),
    in_specs=[_row_spec(D_IN), _row_spec(2), _full_spec(D_IN, D_H),
              _full_spec(1, D_H), _full_spec(D_H, D_H)],
    out_specs=[_row_spec(D_H), _row_spec(HALF), _row_spec(HALF), _row_spec(1)],
    out_shape=[jax.ShapeDtypeStruct((N, D_H), jnp.float32),
               _half_shape, _half_shape,
               jax.ShapeDtypeStruct((N, 1), jnp.float32)],
)

_dense_mid = pl.pallas_call(
    _dense_mid_body,
    grid=(G,),
    in_specs=[_row_spec(HALF), _row_spec(HALF), _row_spec(HALF),
              _row_spec(HALF), _row_spec(1), _row_spec(D_H),
              _full_spec(1, D_H), _full_spec(1, D_H), _full_spec(1, D_H),
              _full_spec(D_H, D_H)],
    out_specs=[_row_spec(D_H), _row_spec(HALF), _row_spec(HALF)],
    out_shape=[jax.ShapeDtypeStruct((N, D_H), jnp.float32),
               _half_shape, _half_shape],
)

_dense_out = pl.pallas_call(
    _dense_out_body,
    grid=(G,),
    in_specs=[_row_spec(HALF), _row_spec(HALF), _row_spec(HALF),
              _row_spec(HALF), _row_spec(1), _row_spec(D_H),
              _full_spec(1, D_H), _full_spec(1, D_H), _full_spec(1, D_H),
              _full_spec(D_H, HALF), _full_spec(1, HALF),
              _full_spec(HALF, 1), _full_spec(1, 1)],
    out_specs=_row_spec(1),
    out_shape=jax.ShapeDtypeStruct((N, 1), jnp.float32),
)


def _conv_agg(hs_lo, hs_hi, src, dst):
    mlo, mhi = _get_sc_msg()(hs_lo, hs_hi, src)
    return _get_sc_agg()(mlo, mhi, dst)


def kernel(x, edge_index, W_in, b_in, Wg1, bg1, g1, be1,
           Wg2, bg2, g2, be2, Wf1, bf1, Wf2, bf2):
    ei = edge_index.astype(jnp.int32)
    src = jnp.concatenate([ei[0], jnp.zeros((PAD,), jnp.int32)])
    dst = jnp.concatenate([ei[1], jnp.full((PAD,), N, jnp.int32)])
    src = src.reshape(CR, CHE)
    dst = dst.reshape(CR, CHE)

    degp = _get_sc_deg()(dst).T

    h0, hs1lo, hs1hi, dinv = _dense_in(
        x, degp, W_in, b_in.reshape(1, D_H), Wg1)
    a1lo, a1hi = _conv_agg(hs1lo, hs1hi, src, dst)
    h1, hs2lo, hs2hi = _dense_mid(
        a1lo, a1hi, hs1lo, hs1hi, dinv, h0, bg1.reshape(1, D_H),
        g1.reshape(1, D_H), be1.reshape(1, D_H), Wg2)
    a2lo, a2hi = _conv_agg(hs2lo, hs2hi, src, dst)
    out2 = _dense_out(
        a2lo, a2hi, hs2lo, hs2hi, dinv, h1, bg2.reshape(1, D_H),
        g2.reshape(1, D_H), be2.reshape(1, D_H), Wf1, bf1.reshape(1, HALF),
        Wf2, bf2.reshape(1, 1))
    return out2[:, 0]
